# baseline (speedup 1.0000x reference)
import functools

import jax
import jax.numpy as jnp
from jax import lax
from jax.experimental import pallas as pl
from jax.experimental.pallas import tpu as pltpu
from jax.experimental.pallas import tpu_sc as plsc

B, H, W, C = 4, 224, 224, 384
L = 16
NPIX = B * H * W
P = 16
CCH = C // L
CPAD = 256


def _sampler_body(img_hbm, xg_hbm, yg_hbm, out_hbm,
                  xv, yv, w_v, i_v,
                  rows_a, rows_b, rows_c, rows_d, out_v,
                  gsem0, gsem1, osem0, osem1):
    info = plsc.get_sparse_core_info()
    nw = info.num_cores * info.num_subcores
    wid = lax.axis_index("s") * info.num_cores + lax.axis_index("c")
    pix_per_w = NPIX // nw
    nchunks = pix_per_w // P
    tile_base = wid * pix_per_w
    row_base = (tile_base // (H * W)) * (H * W)

    xscale = jnp.float32(0.5 * (W - 2))
    yscale = jnp.float32(0.5 * (H - 2))
    gsems = (gsem0, gsem1)
    osems = (osem0, osem1)
    rows = (rows_a, rows_b, rows_c, rows_d)

    pltpu.sync_copy(xg_hbm.at[pl.ds(tile_base, pix_per_w)], xv)
    pltpu.sync_copy(yg_hbm.at[pl.ds(tile_base, pix_per_w)], yv)

    def compute_and_fire(ci, s):
        for k in range(P // L):
            sl = pl.ds(k * L, L)
            gsl = pl.ds(ci * P + k * L, L)
            x = (xv[gsl] + 1.0) * xscale
            y = (yv[gsl] + 1.0) * yscale
            x0 = jnp.clip(x.astype(jnp.int32), 0, W - 1)
            y0 = jnp.clip(y.astype(jnp.int32), 0, H - 1)
            x1 = jnp.minimum(x0 + 1, W - 1)
            y1 = jnp.minimum(y0 + 1, H - 1)
            x0f = x0.astype(jnp.float32)
            x1f = x1.astype(jnp.float32)
            y0f = y0.astype(jnp.float32)
            y1f = y1.astype(jnp.float32)
            wa = (x1f - x) * (y1f - y)
            wb = (x1f - x) * (y - y0f)
            wc = (x - x0f) * (y1f - y)
            wd = (x - x0f) * (y - y0f)
            for t in range(L):
                p = k * L + t
                w_v[s, 0, p, :] = jnp.full((L,), wa[t])
                w_v[s, 1, p, :] = jnp.full((L,), wb[t])
                w_v[s, 2, p, :] = jnp.full((L,), wc[t])
                w_v[s, 3, p, :] = jnp.full((L,), wd[t])
            r0 = row_base + y0 * W
            r1 = row_base + y1 * W
            i_v[s, 0, sl] = r0 + x0
            i_v[s, 1, sl] = r1 + x0
            i_v[s, 2, sl] = r0 + x1
            i_v[s, 3, sl] = r1 + x1
        for j in range(4):
            pltpu.make_async_copy(
                img_hbm.at[i_v.at[s, j]], rows[j].at[s], gsems[s]).start()

    def combine_and_store(ci, s):
        for j in range(4):
            pltpu.make_async_copy(
                img_hbm.at[i_v.at[s, j]], rows[j].at[s], gsems[s]).wait()

        @pl.when(ci > 1)
        def _drain_out():
            pltpu.make_async_copy(
                out_v.at[s], out_hbm.at[pl.ds(tile_base, P)], osems[s]).wait()

        @plsc.parallel_loop(0, P, step=1, unroll=2)
        def pix_body(p):
            wa = w_v[s, 0, p, :]
            wb = w_v[s, 1, p, :]
            wc = w_v[s, 2, p, :]
            wd = w_v[s, 3, p, :]
            himask = jnp.int32(-65536)

            def halves(v):
                lo = lax.bitcast_convert_type(v << 16, jnp.float32)
                hi = lax.bitcast_convert_type(v & himask, jnp.float32)
                return lo, hi

            for c in range(C // (2 * L)):
                cs = pl.ds(c * L, L)
                a_lo, a_hi = halves(rows_a[s, p, cs])
                b_lo, b_hi = halves(rows_b[s, p, cs])
                c_lo, c_hi = halves(rows_c[s, p, cs])
                d_lo, d_hi = halves(rows_d[s, p, cs])
                acc_lo = wa * a_lo + wb * b_lo + wc * c_lo + wd * d_lo
                acc_hi = wa * a_hi + wb * b_hi + wc * c_hi + wd * d_hi
                out_v[s, p, pl.ds(c * 2 * L, L)] = acc_lo
                out_v[s, p, pl.ds(c * 2 * L + L, L)] = acc_hi

        pltpu.make_async_copy(
            out_v.at[s], out_hbm.at[pl.ds(tile_base + ci * P, P)],
            osems[s]).start()

    compute_and_fire(0, 0)

    def pair_body(i, _):
        ci0 = 2 * i
        compute_and_fire(ci0 + 1, 1)
        combine_and_store(ci0, 0)

        @pl.when(ci0 + 2 < nchunks)
        def _prefetch_next():
            compute_and_fire(ci0 + 2, 0)

        combine_and_store(ci0 + 1, 1)
        return _

    lax.fori_loop(0, nchunks // 2, pair_body, 0)
    for s in range(2):
        pltpu.make_async_copy(
            out_v.at[s], out_hbm.at[pl.ds(tile_base, P)], osems[s]).wait()


@jax.jit
def kernel(img, grid):
    img_rows = lax.bitcast_convert_type(
        img.reshape(NPIX, C // (2 * L), 2, L)
        .astype(jnp.bfloat16)
        .transpose(0, 1, 3, 2)
        .reshape(NPIX, C // 2, 2),
        jnp.int32)
    img_rows = jnp.concatenate(
        [img_rows, jnp.zeros((NPIX, CPAD - C // 2), jnp.int32)], axis=1)
    xg = grid[:, 0, :, :].reshape(NPIX)
    yg = grid[:, 1, :, :].reshape(NPIX)

    mesh = plsc.VectorSubcoreMesh(core_axis_name="c", subcore_axis_name="s")
    sampler = functools.partial(
        pl.kernel,
        mesh=mesh,
        out_type=jax.ShapeDtypeStruct((NPIX, C), jnp.float32),
        scratch_types=[
            pltpu.VMEM((NPIX // 32,), jnp.float32),
            pltpu.VMEM((NPIX // 32,), jnp.float32),
            pltpu.VMEM((2, 4, P, L), jnp.float32),
            pltpu.VMEM((2, 4, P), jnp.int32),
            pltpu.VMEM((2, P, CPAD), jnp.int32),
            pltpu.VMEM((2, P, CPAD), jnp.int32),
            pltpu.VMEM((2, P, CPAD), jnp.int32),
            pltpu.VMEM((2, P, CPAD), jnp.int32),
            pltpu.VMEM((2, P, C), jnp.float32),
            pltpu.SemaphoreType.DMA,
            pltpu.SemaphoreType.DMA,
            pltpu.SemaphoreType.DMA,
            pltpu.SemaphoreType.DMA,
        ],
    )(_sampler_body)
    out = sampler(img_rows, xg, yg)
    return out.reshape(B, H, W, C)


# speedup vs baseline: 2.4078x; 2.4078x over previous
import functools

import jax
import jax.numpy as jnp
from jax import lax
from jax.experimental import pallas as pl
from jax.experimental.pallas import tpu as pltpu
from jax.experimental.pallas import tpu_sc as plsc

B, H, W, C = 4, 224, 224, 384
L = 16
NPIX = B * H * W
P = 16
CCH = C // L


def _sampler_body(img_hbm, xg_hbm, yg_hbm, out_hbm,
                  xv, yv, w_v, i_v, rows, out_v,
                  gsem0, gsem1, osem0, osem1):
    info = plsc.get_sparse_core_info()
    nw = info.num_cores * info.num_subcores
    wid = lax.axis_index("s") * info.num_cores + lax.axis_index("c")
    pix_per_w = NPIX // nw
    nchunks = pix_per_w // P
    tile_base = wid * pix_per_w
    row_base = (tile_base // (H * W)) * (H * W)

    xscale = jnp.float32(0.5 * (W - 2))
    yscale = jnp.float32(0.5 * (H - 2))
    gsems = (gsem0, gsem1)
    osems = (osem0, osem1)

    pltpu.sync_copy(xg_hbm.at[pl.ds(tile_base, pix_per_w)], xv)
    pltpu.sync_copy(yg_hbm.at[pl.ds(tile_base, pix_per_w)], yv)

    def compute_and_fire(ci, s):
        for k in range(P // L):
            sl = pl.ds(k * L, L)
            gsl = pl.ds(ci * P + k * L, L)
            x = (xv[gsl] + 1.0) * xscale
            y = (yv[gsl] + 1.0) * yscale
            x0 = jnp.clip(x.astype(jnp.int32), 0, W - 1)
            y0 = jnp.clip(y.astype(jnp.int32), 0, H - 1)
            x1 = jnp.minimum(x0 + 1, W - 1)
            y1 = jnp.minimum(y0 + 1, H - 1)
            x0f = x0.astype(jnp.float32)
            x1f = x1.astype(jnp.float32)
            y0f = y0.astype(jnp.float32)
            y1f = y1.astype(jnp.float32)
            w_v[s, 0, sl] = (x1f - x) * (y1f - y)
            w_v[s, 1, sl] = (x1f - x) * (y - y0f)
            w_v[s, 2, sl] = (x - x0f) * (y1f - y)
            w_v[s, 3, sl] = (x - x0f) * (y - y0f)
            r0 = row_base + y0 * W
            r1 = row_base + y1 * W
            i_v[s, pl.ds(0 * P + k * L, L)] = r0 + x0
            i_v[s, pl.ds(1 * P + k * L, L)] = r1 + x0
            i_v[s, pl.ds(2 * P + k * L, L)] = r0 + x1
            i_v[s, pl.ds(3 * P + k * L, L)] = r1 + x1
        pltpu.make_async_copy(
            img_hbm.at[i_v.at[s]], rows.at[s], gsems[s]).start()

    def combine_and_store(ci, s):
        pltpu.make_async_copy(
            img_hbm.at[i_v.at[s]], rows.at[s], gsems[s]).wait()

        @pl.when(ci > 1)
        def _drain_out():
            pltpu.make_async_copy(
                out_v.at[s], out_hbm.at[pl.ds(tile_base, P)], osems[s]).wait()

        @plsc.parallel_loop(0, P, step=1, unroll=2)
        def pix_body(p):
            pvec = jnp.full((L,), p, dtype=jnp.int32)
            wa = jnp.take_along_axis(w_v[s, 0, :], pvec, axis=0)
            wb = jnp.take_along_axis(w_v[s, 1, :], pvec, axis=0)
            wc = jnp.take_along_axis(w_v[s, 2, :], pvec, axis=0)
            wd = jnp.take_along_axis(w_v[s, 3, :], pvec, axis=0)
            for c in range(CCH):
                cs = pl.ds(c * L, L)
                acc = wa * rows[s, 0 * P + p, cs]
                acc = acc + wb * rows[s, 1 * P + p, cs]
                acc = acc + wc * rows[s, 2 * P + p, cs]
                acc = acc + wd * rows[s, 3 * P + p, cs]
                out_v[s, p, cs] = acc

        pltpu.make_async_copy(
            out_v.at[s], out_hbm.at[pl.ds(tile_base + ci * P, P)],
            osems[s]).start()

    compute_and_fire(0, 0)

    def pair_body(i, _):
        ci0 = 2 * i
        compute_and_fire(ci0 + 1, 1)
        combine_and_store(ci0, 0)

        @pl.when(ci0 + 2 < nchunks)
        def _prefetch_next():
            compute_and_fire(ci0 + 2, 0)

        combine_and_store(ci0 + 1, 1)
        return _

    lax.fori_loop(0, nchunks // 2, pair_body, 0)
    for s in range(2):
        pltpu.make_async_copy(
            out_v.at[s], out_hbm.at[pl.ds(tile_base, P)], osems[s]).wait()


@jax.jit
def kernel(img, grid):
    img_rows = img.reshape(NPIX, C)
    xg = grid[:, 0, :, :].reshape(NPIX)
    yg = grid[:, 1, :, :].reshape(NPIX)

    mesh = plsc.VectorSubcoreMesh(core_axis_name="c", subcore_axis_name="s")
    sampler = functools.partial(
        pl.kernel,
        mesh=mesh,
        out_type=jax.ShapeDtypeStruct((NPIX, C), jnp.float32),
        scratch_types=[
            pltpu.VMEM((NPIX // 32,), jnp.float32),
            pltpu.VMEM((NPIX // 32,), jnp.float32),
            pltpu.VMEM((2, 4, P), jnp.float32),
            pltpu.VMEM((2, 4 * P), jnp.int32),
            pltpu.VMEM((2, 4 * P, C), jnp.float32),
            pltpu.VMEM((2, P, C), jnp.float32),
            pltpu.SemaphoreType.DMA,
            pltpu.SemaphoreType.DMA,
            pltpu.SemaphoreType.DMA,
            pltpu.SemaphoreType.DMA,
        ],
    )(_sampler_body)
    out = sampler(img_rows, xg, yg)
    return out.reshape(B, H, W, C)


# speedup vs baseline: 3.5584x; 1.4779x over previous
import functools

import jax
import jax.numpy as jnp
from jax import lax
from jax.experimental import pallas as pl
from jax.experimental.pallas import tpu as pltpu
from jax.experimental.pallas import tpu_sc as plsc

B, H, W, C = 4, 224, 224, 384
L = 16
NPIX = B * H * W
P = 16
CCH = C // L


def _sampler_body(img_hbm, xg_hbm, yg_hbm, out_hbm,
                  xv, yv, w_v, i_v,
                  rows_a, rows_b, rows_c, rows_d, out_v,
                  gsem0, gsem1, osem0, osem1):
    info = plsc.get_sparse_core_info()
    nw = info.num_cores * info.num_subcores
    wid = lax.axis_index("s") * info.num_cores + lax.axis_index("c")
    pix_per_w = NPIX // nw
    nchunks = pix_per_w // P
    tile_base = wid * pix_per_w
    row_base = (tile_base // (H * W)) * (H * W)

    xscale = jnp.float32(0.5 * (W - 2))
    yscale = jnp.float32(0.5 * (H - 2))
    gsems = (gsem0, gsem1)
    osems = (osem0, osem1)
    rows = (rows_a, rows_b, rows_c, rows_d)

    pltpu.sync_copy(xg_hbm.at[pl.ds(tile_base, pix_per_w)], xv)
    pltpu.sync_copy(yg_hbm.at[pl.ds(tile_base, pix_per_w)], yv)

    def compute_and_fire(ci, s):
        for k in range(P // L):
            sl = pl.ds(k * L, L)
            gsl = pl.ds(ci * P + k * L, L)
            x = (xv[gsl] + 1.0) * xscale
            y = (yv[gsl] + 1.0) * yscale
            x0 = jnp.clip(x.astype(jnp.int32), 0, W - 1)
            y0 = jnp.clip(y.astype(jnp.int32), 0, H - 1)
            x1 = jnp.minimum(x0 + 1, W - 1)
            y1 = jnp.minimum(y0 + 1, H - 1)
            x0f = x0.astype(jnp.float32)
            x1f = x1.astype(jnp.float32)
            y0f = y0.astype(jnp.float32)
            y1f = y1.astype(jnp.float32)
            wa = (x1f - x) * (y1f - y)
            wb = (x1f - x) * (y - y0f)
            wc = (x - x0f) * (y1f - y)
            wd = (x - x0f) * (y - y0f)
            for t in range(L):
                p = k * L + t
                w_v[s, 0, p, :] = jnp.full((L,), wa[t])
                w_v[s, 1, p, :] = jnp.full((L,), wb[t])
                w_v[s, 2, p, :] = jnp.full((L,), wc[t])
                w_v[s, 3, p, :] = jnp.full((L,), wd[t])
            r0 = row_base + y0 * W
            r1 = row_base + y1 * W
            i_v[s, 0, sl] = r0 + x0
            i_v[s, 1, sl] = r1 + x0
            i_v[s, 2, sl] = r0 + x1
            i_v[s, 3, sl] = r1 + x1
        for j in range(4):
            pltpu.make_async_copy(
                img_hbm.at[i_v.at[s, j]], rows[j].at[s], gsems[s]).start()

    def combine_and_store(ci, s):
        for j in range(4):
            pltpu.make_async_copy(
                img_hbm.at[i_v.at[s, j]], rows[j].at[s], gsems[s]).wait()

        @pl.when(ci > 1)
        def _drain_out():
            pltpu.make_async_copy(
                out_v.at[s], out_hbm.at[pl.ds(tile_base, P)], osems[s]).wait()

        @plsc.parallel_loop(0, P, step=1, unroll=2)
        def pix_body(p):
            wa = w_v[s, 0, p, :]
            wb = w_v[s, 1, p, :]
            wc = w_v[s, 2, p, :]
            wd = w_v[s, 3, p, :]
            for c in range(CCH):
                cs = pl.ds(c * L, L)
                acc = wa * rows_a[s, p, cs]
                acc = acc + wb * rows_b[s, p, cs]
                acc = acc + wc * rows_c[s, p, cs]
                acc = acc + wd * rows_d[s, p, cs]
                out_v[s, p, cs] = acc

        pltpu.make_async_copy(
            out_v.at[s], out_hbm.at[pl.ds(tile_base + ci * P, P)],
            osems[s]).start()

    compute_and_fire(0, 0)

    def pair_body(i, _):
        ci0 = 2 * i
        compute_and_fire(ci0 + 1, 1)
        combine_and_store(ci0, 0)

        @pl.when(ci0 + 2 < nchunks)
        def _prefetch_next():
            compute_and_fire(ci0 + 2, 0)

        combine_and_store(ci0 + 1, 1)
        return _

    lax.fori_loop(0, nchunks // 2, pair_body, 0)
    for s in range(2):
        pltpu.make_async_copy(
            out_v.at[s], out_hbm.at[pl.ds(tile_base, P)], osems[s]).wait()


@jax.jit
def kernel(img, grid):
    img_rows = img.reshape(NPIX, C)
    xg = grid[:, 0, :, :].reshape(NPIX)
    yg = grid[:, 1, :, :].reshape(NPIX)

    mesh = plsc.VectorSubcoreMesh(core_axis_name="c", subcore_axis_name="s")
    sampler = functools.partial(
        pl.kernel,
        mesh=mesh,
        out_type=jax.ShapeDtypeStruct((NPIX, C), jnp.float32),
        scratch_types=[
            pltpu.VMEM((NPIX // 32,), jnp.float32),
            pltpu.VMEM((NPIX // 32,), jnp.float32),
            pltpu.VMEM((2, 4, P, L), jnp.float32),
            pltpu.VMEM((2, 4, P), jnp.int32),
            pltpu.VMEM((2, P, C), jnp.float32),
            pltpu.VMEM((2, P, C), jnp.float32),
            pltpu.VMEM((2, P, C), jnp.float32),
            pltpu.VMEM((2, P, C), jnp.float32),
            pltpu.VMEM((2, P, C), jnp.float32),
            pltpu.SemaphoreType.DMA,
            pltpu.SemaphoreType.DMA,
            pltpu.SemaphoreType.DMA,
            pltpu.SemaphoreType.DMA,
        ],
    )(_sampler_body)
    out = sampler(img_rows, xg, yg)
    return out.reshape(B, H, W, C)
